# baseline (speedup 1.0000x reference)
import functools

import jax
import jax.numpy as jnp
from jax import lax
from jax.experimental import pallas as pl
from jax.experimental.pallas import tpu as pltpu, tpu_sc as plsc

_TOKENS = 32768
_DIM = 768
_EXPERTS = 8
_BLOCK = 4096

_NC = 2
_NS = 16
_NW = _NC * _NS
_CHUNK = _TOKENS // _NW
_LANES = 16
_NEG = -3.0e38


def _logits_body(x_ref, w_ref, b_ref, lt_ref):
    xb = x_ref[...]
    lt = jax.lax.dot_general(
        w_ref[...], xb, (((0,), (1,)), ((), ())),
        preferred_element_type=jnp.float32,
    )
    lt_ref[...] = lt + b_ref[...]


def _tc_logits(x, W_gate, b2d):
    n_blocks = _TOKENS // _BLOCK
    grid_spec = pl.GridSpec(
        grid=(n_blocks,),
        in_specs=[
            pl.BlockSpec((_BLOCK, _DIM), lambda i: (i, 0)),
            pl.BlockSpec((_DIM, _EXPERTS), lambda i: (0, 0)),
            pl.BlockSpec((_EXPERTS, 1), lambda i: (0, 0)),
        ],
        out_specs=[
            pl.BlockSpec((_EXPERTS, _BLOCK), lambda i: (0, i)),
        ],
    )
    (lt,) = pl.pallas_call(
        _logits_body,
        grid_spec=grid_spec,
        out_shape=[jax.ShapeDtypeStruct((_EXPERTS, _TOKENS), jnp.float32)],
    )(x, W_gate, b2d)
    return lt


def _sc_route_body(lt_hbm, it_hbm, vt_hbm, chunk, oi, ov, sem):
    wid = lax.axis_index("s") * _NC + lax.axis_index("c")
    base = wid * _CHUNK
    copies = [
        pltpu.async_copy(
            lt_hbm.at[e, pl.ds(base, _CHUNK)], chunk.at[e], sem
        )
        for e in range(_EXPERTS)
    ]
    for c in copies:
        c.wait()

    one = jnp.full((_LANES,), 1.0, jnp.float32)
    ninf = jnp.full((_LANES,), _NEG, jnp.float32)
    econst = [jnp.full((_LANES,), e, jnp.int32) for e in range(_EXPERTS)]
    for j in range(_CHUNK // _LANES):
        off = j * _LANES
        le = [chunk[e, pl.ds(off, _LANES)] for e in range(_EXPERTS)]
        m1 = le[0]
        for e in range(1, _EXPERTS):
            m1 = jnp.maximum(m1, le[e])
        i1 = econst[_EXPERTS - 1]
        for e in range(_EXPERTS - 2, -1, -1):
            i1 = jnp.where(le[e] == m1, econst[e], i1)
        z = jnp.exp(le[0] - m1)
        for e in range(1, _EXPERTS):
            z = z + jnp.exp(le[e] - m1)
        cand = [
            jnp.where(i1 == econst[e], ninf, le[e]) for e in range(_EXPERTS)
        ]
        m2 = cand[0]
        for e in range(1, _EXPERTS):
            m2 = jnp.maximum(m2, cand[e])
        i2 = econst[_EXPERTS - 1]
        for e in range(_EXPERTS - 2, -1, -1):
            i2 = jnp.where(cand[e] == m2, econst[e], i2)
        inv_z = one / z
        v2 = jnp.exp(m2 - m1) * inv_z
        oi[0, pl.ds(off, _LANES)] = i1
        oi[1, pl.ds(off, _LANES)] = i2
        ov[0, pl.ds(off, _LANES)] = inv_z
        ov[1, pl.ds(off, _LANES)] = v2

    for r in range(2):
        pltpu.sync_copy(oi.at[r], it_hbm.at[r, pl.ds(base, _CHUNK)])
        pltpu.sync_copy(ov.at[r], vt_hbm.at[r, pl.ds(base, _CHUNK)])


def _sc_route(lt):
    mesh = plsc.VectorSubcoreMesh(core_axis_name="c", subcore_axis_name="s")
    kfn = functools.partial(
        pl.kernel,
        mesh=mesh,
        out_type=[
            jax.ShapeDtypeStruct((2, _TOKENS), jnp.int32),
            jax.ShapeDtypeStruct((2, _TOKENS), jnp.float32),
        ],
        scratch_types=[
            pltpu.VMEM((_EXPERTS, _CHUNK), jnp.float32),
            pltpu.VMEM((2, _CHUNK), jnp.int32),
            pltpu.VMEM((2, _CHUNK), jnp.float32),
            pltpu.SemaphoreType.DMA,
        ],
    )(_sc_route_body)
    return kfn(lt)


def kernel(x, W_gate, b_gate):
    b2d = b_gate.reshape(_EXPERTS, 1)
    lt = _tc_logits(x, W_gate, b2d)
    idx_t, val_t = _sc_route(lt)
    return idx_t.T, val_t.T


# speedup vs baseline: 1.6318x; 1.6318x over previous
import jax
import jax.numpy as jnp
from jax.experimental import pallas as pl
from jax.experimental.pallas import tpu as pltpu

_TOKENS = 32768
_DIM = 768
_EXPERTS = 8
_BLOCK = 4096


def _gate_body(x_ref, w_ref, b_ref, idx_ref, val_ref):
    xb = x_ref[...]
    lt = jax.lax.dot_general(
        w_ref[...], xb, (((0,), (1,)), ((), ())),
        preferred_element_type=jnp.float32,
    )
    lt = lt + b_ref[...]
    ids = jax.lax.broadcasted_iota(jnp.int32, lt.shape, 0)
    m1 = jnp.max(lt, axis=0, keepdims=True)
    z = jnp.sum(jnp.exp(lt - m1), axis=0, keepdims=True)
    i1 = jnp.min(jnp.where(lt == m1, ids, _EXPERTS), axis=0, keepdims=True)
    masked = jnp.where(ids == i1, -jnp.inf, lt)
    m2 = jnp.max(masked, axis=0, keepdims=True)
    i2 = jnp.min(jnp.where(masked == m2, ids, _EXPERTS), axis=0, keepdims=True)
    inv_z = 1.0 / z
    v2 = jnp.exp(m2 - m1) * inv_z
    idx_ref[...] = jnp.concatenate([i1, i2], axis=0)
    val_ref[...] = jnp.concatenate([inv_z, v2], axis=0)


def kernel(x, W_gate, b_gate):
    n_blocks = _TOKENS // _BLOCK
    b2d = b_gate.reshape(_EXPERTS, 1)
    grid_spec = pl.GridSpec(
        grid=(n_blocks,),
        in_specs=[
            pl.BlockSpec((_BLOCK, _DIM), lambda i: (i, 0)),
            pl.BlockSpec((_DIM, _EXPERTS), lambda i: (0, 0)),
            pl.BlockSpec((_EXPERTS, 1), lambda i: (0, 0)),
        ],
        out_specs=[
            pl.BlockSpec((2, _BLOCK), lambda i: (0, i)),
            pl.BlockSpec((2, _BLOCK), lambda i: (0, i)),
        ],
    )
    idx_t, val_t = pl.pallas_call(
        _gate_body,
        grid_spec=grid_spec,
        compiler_params=pltpu.CompilerParams(
            dimension_semantics=("parallel",),
        ),
        out_shape=[
            jax.ShapeDtypeStruct((2, _TOKENS), jnp.int32),
            jax.ShapeDtypeStruct((2, _TOKENS), jnp.float32),
        ],
    )(x, W_gate, b2d)
    return idx_t.T, val_t.T
